# Initial kernel scaffold; baseline (speedup 1.0000x reference)
import jax
import jax.numpy as jnp
from jax.experimental import pallas as pl


def kernel(h0, h1, tdW_conv, tdb_conv, tdW_fus, tdb_fus, td_conv_w, td_topDown_w, td_gamma, td_beta, buW_conv, bub_conv, buW_fus, bub_fus, bu_conv_w, bu_bottomUp_w, bu_gamma, bu_beta, edge_index0, edge_index1):
    raise NotImplementedError("write your pallas kernel here")


# speedup vs baseline: 1.4821x; 1.4821x over previous
import jax
import jax.numpy as jnp
from jax.experimental import pallas as pl

_D = 128
_BLK = 2000


def _mm2_body(xa_ref, xb_ref, wa_ref, wb_ref, ns_ref, oa_ref, ob_ref):
    ns = ns_ref[...]
    oa_ref[...] = jnp.dot(xa_ref[...], wa_ref[...],
                          preferred_element_type=jnp.float32) * ns
    ob_ref[...] = jnp.dot(xb_ref[...], wb_ref[...],
                          preferred_element_type=jnp.float32) * ns


def _mm2(xa, xb, wa, wb, ns):
    n = xa.shape[0]
    bs_x = pl.BlockSpec((_BLK, _D), lambda i: (i, 0))
    bs_w = pl.BlockSpec((_D, _D), lambda i: (0, 0))
    bs_n = pl.BlockSpec((_BLK, 1), lambda i: (i, 0))
    return pl.pallas_call(
        _mm2_body,
        grid=(n // _BLK,),
        in_specs=[bs_x, bs_x, bs_w, bs_w, bs_n],
        out_specs=[bs_x, bs_x],
        out_shape=[jax.ShapeDtypeStruct((n, _D), jnp.float32)] * 2,
    )(xa, xb, wa, wb, ns)


def _post_body(aa_ref, ha_ref, ab_ref, hb_ref, nd_ref,
               bca_ref, wca_ref, bcb_ref, wcb_ref, g_ref, be_ref, o_ref):
    nd = nd_ref[...]
    conv = ((aa_ref[...] + ha_ref[...]) * nd + bca_ref[...]) * wca_ref[...]
    fus = ((ab_ref[...] + hb_ref[...]) * nd + bcb_ref[...]) * wcb_ref[...]
    z = conv + fus
    mu = jnp.mean(z, axis=-1, keepdims=True)
    var = jnp.mean((z - mu) ** 2, axis=-1, keepdims=True)
    y = (z - mu) / jnp.sqrt(var + 1e-5) * g_ref[...] + be_ref[...]
    o_ref[...] = jnp.maximum(y, 0.0)


def _post(aa, ha, ab, hb, nd, bc, wc, bf, wf, gamma, beta):
    n = aa.shape[0]
    bs_x = pl.BlockSpec((_BLK, _D), lambda i: (i, 0))
    bs_n = pl.BlockSpec((_BLK, 1), lambda i: (i, 0))
    bs_p = pl.BlockSpec((1, _D), lambda i: (0, 0))
    return pl.pallas_call(
        _post_body,
        grid=(n // _BLK,),
        in_specs=[bs_x, bs_x, bs_x, bs_x, bs_n,
                  bs_p, bs_p, bs_p, bs_p, bs_p, bs_p],
        out_specs=bs_x,
        out_shape=jax.ShapeDtypeStruct((n, _D), jnp.float32),
    )(aa, ha, ab, hb, nd,
      bc.reshape(1, _D), wc.reshape(1, _D),
      bf.reshape(1, _D), wf.reshape(1, _D),
      gamma.reshape(1, _D), beta.reshape(1, _D))


def _level(x_conv, x_fus, src, dst, n, Wc, bc, Wf, bf, wc, wf, gamma, beta):
    deg_out = (jnp.bincount(src, length=n) + 1).astype(jnp.float32)
    deg_in = (jnp.bincount(dst, length=n) + 1).astype(jnp.float32)
    ns = (deg_out ** -0.5).reshape(n, 1)
    nd = (deg_in ** -0.5).reshape(n, 1)
    ha, hb = _mm2(x_conv, x_fus, Wc, Wf, ns)
    h_cat = jnp.concatenate([ha, hb], axis=1)
    agg = jax.ops.segment_sum(h_cat[src], dst, num_segments=n)
    return _post(agg[:, :_D], ha, agg[:, _D:], hb, nd,
                 bc, wc, bf, wf, gamma, beta)


def kernel(h0, h1, tdW_conv, tdb_conv, tdW_fus, tdb_fus, td_conv_w,
           td_topDown_w, td_gamma, td_beta, buW_conv, bub_conv, buW_fus,
           bub_fus, bu_conv_w, bu_bottomUp_w, bu_gamma, bu_beta,
           edge_index0, edge_index1):
    n0 = h0.shape[0]
    n1 = h1.shape[0]
    src0, dst0 = edge_index0[0], edge_index0[1]
    src1, dst1 = edge_index1[0], edge_index1[1]
    inc_h1 = jax.ops.segment_sum(h1, dst0, num_segments=n0)
    r0 = _level(h0, inc_h1, src0, dst0, n0, tdW_conv, tdb_conv,
                tdW_fus, tdb_fus, td_conv_w, td_topDown_w, td_gamma, td_beta)
    incT_h0 = h0[dst0]
    r1 = _level(h1, incT_h0, src1, dst1, n1, buW_conv, bub_conv,
                buW_fus, bub_fus, bu_conv_w, bu_bottomUp_w, bu_gamma, bu_beta)
    return (r0, r1)


# speedup vs baseline: 1.4895x; 1.0050x over previous
import jax
import jax.numpy as jnp
from jax.experimental import pallas as pl

_D = 128
def _blk(n):
    return 4000 if n % 4000 == 0 else 2000


def _mm2_body(xa_ref, xb_ref, wa_ref, wb_ref, ns_ref, oa_ref, ob_ref):
    ns = ns_ref[...]
    oa_ref[...] = jnp.dot(xa_ref[...], wa_ref[...],
                          preferred_element_type=jnp.float32) * ns
    ob_ref[...] = jnp.dot(xb_ref[...], wb_ref[...],
                          preferred_element_type=jnp.float32) * ns


def _mm2(xa, xb, wa, wb, ns):
    n = xa.shape[0]
    blk = _blk(n)
    bs_x = pl.BlockSpec((blk, _D), lambda i: (i, 0))
    bs_w = pl.BlockSpec((_D, _D), lambda i: (0, 0))
    bs_n = pl.BlockSpec((blk, 1), lambda i: (i, 0))
    return pl.pallas_call(
        _mm2_body,
        grid=(n // blk,),
        in_specs=[bs_x, bs_x, bs_w, bs_w, bs_n],
        out_specs=[bs_x, bs_x],
        out_shape=[jax.ShapeDtypeStruct((n, _D), jnp.float32)] * 2,
    )(xa, xb, wa, wb, ns)


def _post_body(aa_ref, ha_ref, ab_ref, hb_ref, nd_ref,
               bca_ref, wca_ref, bcb_ref, wcb_ref, g_ref, be_ref, o_ref):
    nd = nd_ref[...]
    conv = ((aa_ref[...] + ha_ref[...]) * nd + bca_ref[...]) * wca_ref[...]
    fus = ((ab_ref[...] + hb_ref[...]) * nd + bcb_ref[...]) * wcb_ref[...]
    z = conv + fus
    mu = jnp.mean(z, axis=-1, keepdims=True)
    var = jnp.mean((z - mu) ** 2, axis=-1, keepdims=True)
    y = (z - mu) / jnp.sqrt(var + 1e-5) * g_ref[...] + be_ref[...]
    o_ref[...] = jnp.maximum(y, 0.0)


def _post(aa, ha, ab, hb, nd, bc, wc, bf, wf, gamma, beta):
    n = aa.shape[0]
    blk = _blk(n)
    bs_x = pl.BlockSpec((blk, _D), lambda i: (i, 0))
    bs_n = pl.BlockSpec((blk, 1), lambda i: (i, 0))
    bs_p = pl.BlockSpec((1, _D), lambda i: (0, 0))
    return pl.pallas_call(
        _post_body,
        grid=(n // blk,),
        in_specs=[bs_x, bs_x, bs_x, bs_x, bs_n,
                  bs_p, bs_p, bs_p, bs_p, bs_p, bs_p],
        out_specs=bs_x,
        out_shape=jax.ShapeDtypeStruct((n, _D), jnp.float32),
    )(aa, ha, ab, hb, nd,
      bc.reshape(1, _D), wc.reshape(1, _D),
      bf.reshape(1, _D), wf.reshape(1, _D),
      gamma.reshape(1, _D), beta.reshape(1, _D))


def _level(x_conv, x_fus, src, dst, n, Wc, bc, Wf, bf, wc, wf, gamma, beta):
    deg_out = (jnp.bincount(src, length=n) + 1).astype(jnp.float32)
    deg_in = (jnp.bincount(dst, length=n) + 1).astype(jnp.float32)
    ns = (deg_out ** -0.5).reshape(n, 1)
    nd = (deg_in ** -0.5).reshape(n, 1)
    ha, hb = _mm2(x_conv, x_fus, Wc, Wf, ns)
    h_cat = jnp.concatenate([ha, hb], axis=1)
    agg = jax.ops.segment_sum(h_cat[src], dst, num_segments=n)
    return _post(agg[:, :_D], ha, agg[:, _D:], hb, nd,
                 bc, wc, bf, wf, gamma, beta)


def kernel(h0, h1, tdW_conv, tdb_conv, tdW_fus, tdb_fus, td_conv_w,
           td_topDown_w, td_gamma, td_beta, buW_conv, bub_conv, buW_fus,
           bub_fus, bu_conv_w, bu_bottomUp_w, bu_gamma, bu_beta,
           edge_index0, edge_index1):
    n0 = h0.shape[0]
    n1 = h1.shape[0]
    src0, dst0 = edge_index0[0], edge_index0[1]
    src1, dst1 = edge_index1[0], edge_index1[1]
    inc_h1 = jax.ops.segment_sum(h1, dst0, num_segments=n0)
    r0 = _level(h0, inc_h1, src0, dst0, n0, tdW_conv, tdb_conv,
                tdW_fus, tdb_fus, td_conv_w, td_topDown_w, td_gamma, td_beta)
    incT_h0 = h0[dst0]
    r1 = _level(h1, incT_h0, src1, dst1, n1, buW_conv, bub_conv,
                buW_fus, bub_fus, bu_conv_w, bu_bottomUp_w, bu_gamma, bu_beta)
    return (r0, r1)
